# baseline (speedup 1.0000x reference)
import functools

import jax
import jax.numpy as jnp
from jax import lax
from jax.experimental import pallas as pl
from jax.experimental.pallas import tpu as pltpu
from jax.experimental.pallas import tpu_sc as plsc

_NUM_IMG = 100000
_OUT_CH = 64
_BATCH = 1024

_NC = 2
_NS = 16
_NW = _NS
_HALF = _BATCH // 2
_BLOC = _HALF // _NW
_LANES = 16
_TILE = 128
_RND = 4
_NROUNDS = _BLOC // _RND
_NBANK = 3
_NBUF = _NBANK * _RND

_mesh = plsc.VectorSubcoreMesh(
    core_axis_name="c", subcore_axis_name="s", num_cores=1
)


@functools.partial(
    pl.kernel,
    mesh=_mesh,
    out_type=jax.ShapeDtypeStruct((_HALF, _OUT_CH), jnp.float32),
    compiler_params=pltpu.CompilerParams(
        needs_layout_passes=False, skip_device_barrier=True
    ),
    scratch_types=[
        pltpu.VMEM((_BLOC + _LANES,), jnp.int32),
        pltpu.VMEM((_OUT_CH,), jnp.int32),
        pltpu.VMEM((_NBUF, _OUT_CH, _TILE), jnp.float32),
        pltpu.VMEM((_BLOC, _OUT_CH), jnp.float32),
        pltpu.SemaphoreType.DMA,
        pltpu.SemaphoreType.DMA,
    ],
)
def _gather_kernel(x_hbm, w_hbm, out_hbm, x_v, ch_v, blk_v, rows_v, sem, wsem):
    wid = lax.axis_index("s")
    base = wid * _BLOC
    pltpu.sync_copy(x_hbm.at[pl.ds(base, _BLOC)], x_v.at[pl.ds(0, _BLOC)])

    lane = lax.iota(jnp.int32, _LANES)
    for cb in range(_OUT_CH // _LANES):
        ch_v[pl.ds(cb * _LANES, _LANES)] = lane + cb * _LANES

    def _fire(xs, slot):
        col = xs & (_TILE - 1)
        start = pl.multiple_of(xs - col, _TILE)
        pltpu.async_copy(
            w_hbm.at[ch_v, pl.ds(start, _TILE)], blk_v.at[slot], sem
        )

    def _wait(slot):
        pltpu.make_async_copy(
            w_hbm.at[pl.ds(0, _OUT_CH), pl.ds(0, _TILE)], blk_v.at[slot], sem
        ).wait()

    def _extract(b, slot, xs):
        colvec = lane * 0 + (xs & (_TILE - 1))
        for cb in range(_OUT_CH // _LANES):
            vals = plsc.load_gather(
                blk_v.at[slot], [lane + cb * _LANES, colvec]
            )
            rows_v[b, pl.ds(cb * _LANES, _LANES)] = vals

    xv0 = x_v[pl.ds(0, _LANES)]
    for j in range(_RND):
        _fire(xv0[j], j)
    for j in range(_RND):
        _fire(xv0[_RND + j], _RND + j)

    def _body(r, carry):
        b0 = r * _RND
        slot0 = lax.rem(r, _NBANK) * _RND
        nslot0 = lax.rem(r + 2, _NBANK) * _RND
        xv = x_v[pl.ds(b0, _LANES)]

        for j in range(_RND):
            _wait(slot0 + j)

        @pl.when(r < _NROUNDS - 2)
        def _refill():
            for j in range(_RND):
                _fire(xv[2 * _RND + j], nslot0 + j)

        for j in range(_RND):
            _extract(b0 + j, slot0 + j, xv[j])

        @pl.when(lax.rem(r, 2) == 1)
        def _flush():
            w0 = b0 - _RND
            pltpu.async_copy(
                rows_v.at[pl.ds(w0, 2 * _RND)],
                out_hbm.at[pl.ds(pl.multiple_of(base + w0, 8), 2 * _RND)],
                wsem,
            )
        return carry

    lax.fori_loop(0, _NROUNDS, _body, 0)

    for _ in range(_NROUNDS // 2):
        pltpu.make_async_copy(
            rows_v.at[pl.ds(0, 2 * _RND)],
            out_hbm.at[pl.ds(base, 2 * _RND)],
            wsem,
        ).wait()


def kernel(x, W):
    xi = x.astype(jnp.int32)
    y0 = _gather_kernel(xi[:_HALF], W)
    y1 = _gather_kernel(xi[_HALF:], W)
    y = jnp.concatenate([y0, y1], axis=0)
    return y[:, :, None, None]


# speedup vs baseline: 1.4237x; 1.4237x over previous
import functools

import jax
import jax.numpy as jnp
from jax import lax
from jax.experimental import pallas as pl
from jax.experimental.pallas import tpu as pltpu
from jax.experimental.pallas import tpu_sc as plsc

_NUM_IMG = 100000
_OUT_CH = 64
_BATCH = 1024

_NC = 2
_NS = 16
_NW = _NC * _NS
_BLOC = _BATCH // _NW
_LANES = 16
_TILE = 128
_RND = 4
_NROUNDS = _BLOC // _RND
_NBANK = 3
_NBUF = _NBANK * _RND

_mesh = plsc.VectorSubcoreMesh(core_axis_name="c", subcore_axis_name="s")


@functools.partial(
    pl.kernel,
    mesh=_mesh,
    out_type=jax.ShapeDtypeStruct((_BATCH, _OUT_CH), jnp.float32),
    compiler_params=pltpu.CompilerParams(needs_layout_passes=False),
    scratch_types=[
        pltpu.VMEM((_BLOC + _LANES,), jnp.int32),
        pltpu.VMEM((_OUT_CH,), jnp.int32),
        pltpu.VMEM((_NBUF, _OUT_CH, _TILE), jnp.float32),
        pltpu.VMEM((_BLOC, _OUT_CH), jnp.float32),
        pltpu.SemaphoreType.DMA,
        pltpu.SemaphoreType.DMA,
    ],
)
def _gather_kernel(x_hbm, w_hbm, out_hbm, x_v, ch_v, blk_v, rows_v, sem, wsem):
    wid = lax.axis_index("s") * _NC + lax.axis_index("c")
    base = wid * _BLOC
    pltpu.sync_copy(x_hbm.at[pl.ds(base, _BLOC)], x_v.at[pl.ds(0, _BLOC)])

    lane = lax.iota(jnp.int32, _LANES)
    for cb in range(_OUT_CH // _LANES):
        ch_v[pl.ds(cb * _LANES, _LANES)] = lane + cb * _LANES

    def _fire(xs, slot):
        col = xs & (_TILE - 1)
        start = pl.multiple_of(xs - col, _TILE)
        pltpu.async_copy(
            w_hbm.at[ch_v, pl.ds(start, _TILE)], blk_v.at[slot], sem
        )

    def _wait(slot):
        pltpu.make_async_copy(
            w_hbm.at[pl.ds(0, _OUT_CH), pl.ds(0, _TILE)], blk_v.at[slot], sem
        ).wait()

    def _extract(b, slot, xs):
        colvec = lane * 0 + (xs & (_TILE - 1))
        for cb in range(_OUT_CH // _LANES):
            vals = plsc.load_gather(
                blk_v.at[slot], [lane + cb * _LANES, colvec]
            )
            rows_v[b, pl.ds(cb * _LANES, _LANES)] = vals

    xv0 = x_v[pl.ds(0, _LANES)]
    for j in range(_RND):
        _fire(xv0[j], j)
    for j in range(_RND):
        _fire(xv0[_RND + j], _RND + j)

    def _body(r, carry):
        b0 = r * _RND
        slot0 = lax.rem(r, _NBANK) * _RND
        nslot0 = lax.rem(r + 2, _NBANK) * _RND
        xv = x_v[pl.ds(b0, _LANES)]

        for j in range(_RND):
            _wait(slot0 + j)

        @pl.when(r < _NROUNDS - 2)
        def _refill():
            for j in range(_RND):
                _fire(xv[2 * _RND + j], nslot0 + j)

        for j in range(_RND):
            _extract(b0 + j, slot0 + j, xv[j])

        @pl.when(lax.rem(r, 2) == 1)
        def _flush():
            w0 = b0 - _RND
            pltpu.async_copy(
                rows_v.at[pl.ds(w0, 2 * _RND)],
                out_hbm.at[pl.ds(pl.multiple_of(base + w0, 8), 2 * _RND)],
                wsem,
            )
        return carry

    lax.fori_loop(0, _NROUNDS, _body, 0)

    for _ in range(_NROUNDS // 2):
        pltpu.make_async_copy(
            rows_v.at[pl.ds(0, 2 * _RND)],
            out_hbm.at[pl.ds(base, 2 * _RND)],
            wsem,
        ).wait()


def kernel(x, W):
    xi = x.astype(jnp.int32)
    y = _gather_kernel(xi, W)
    return y[:, :, None, None]
